# baseline (speedup 1.0000x reference)
import jax
import jax.numpy as jnp
from jax import lax
from jax.experimental import pallas as pl
from jax.experimental.pallas import tpu as pltpu
from jax.experimental.pallas import tpu_sc as plsc

N = 10000
D = 128
E = 320000
E2 = 2 * E
NC, NS = 2, 16
NW = NC * NS
CHUNK_ROWS = 2
EPW = 20480
E2P = NW * EPW
ROWS = E2P // 128
RPW = EPW // 128
NP = 10008
NPT = 632
NPD = NS * NPT

_MESH = plsc.VectorSubcoreMesh(core_axis_name="c", subcore_axis_name="s",
                               num_cores=NC, num_subcores=NS)


R0 = 288
R1 = 32
assert 16 * (R0 + R1) == ROWS and R0 % 2 == 0 and R1 % 2 == 0


def _sc_agg_body(tgt_hbm, nbr_hbm, feats_hbm, zer_hbm, agg_out,
                 idx_t, idx_n, rows_v, agg_sh, sem):
    cid = lax.axis_index("c")
    sid = lax.axis_index("s")

    aoff = jnp.minimum(sid * NPT, NP - NPT)
    pltpu.sync_copy(zer_hbm, agg_sh.at[pl.ds(aoff, NPT)])
    plsc.subcore_barrier()

    base = jnp.where(cid == 0, sid * R0, NS * R0 + sid * R1)
    nsteps = jnp.where(cid == 0, R0 // 2, R1 // 2)

    def step(g, carry):
        r0 = base + g * 2
        pltpu.sync_copy(tgt_hbm.at[pl.ds(r0, 2)], idx_t)
        pltpu.sync_copy(nbr_hbm.at[pl.ds(r0, 2)], idx_n)
        for j in range(2):
            pltpu.async_copy(feats_hbm.at[idx_n.at[j]],
                             rows_v.at[pl.ds(j * 128, 128)], sem).wait()
        for j in range(2):
            pltpu.sync_copy(rows_v.at[pl.ds(j * 128, 128)],
                            agg_sh.at[idx_t.at[j]], add=True)
        return carry

    lax.fori_loop(0, nsteps, step, 0)
    plsc.subcore_barrier()

    pltpu.sync_copy(agg_sh.at[pl.ds(aoff, NPT)],
                    agg_out.at[pl.ds(cid * NP + aoff, NPT)])


_sc_agg = pl.kernel(
    _sc_agg_body,
    out_type=jax.ShapeDtypeStruct((NC * NP, D), jnp.float32),
    mesh=_MESH,
    scratch_types=[
        pltpu.VMEM((2, 128), jnp.int32),
        pltpu.VMEM((2, 128), jnp.int32),
        pltpu.VMEM((2 * 128, D), jnp.float32),
        pltpu.VMEM_SHARED((NP, D), jnp.float32),
        pltpu.SemaphoreType.DMA,
    ],
)


def _sc_deg_body(tgt_hbm, zer_hbm, one_hbm, deg_out,
                 idx_t, ones_v, deg_sh, sem):
    cid = lax.axis_index("c")
    sid = lax.axis_index("s")
    wid = sid * NC + cid

    pltpu.sync_copy(zer_hbm, deg_sh.at[pl.ds(sid * NPT, NPT)])
    pltpu.sync_copy(one_hbm, ones_v)
    plsc.subcore_barrier()

    base = wid * RPW

    def step(g, carry):
        r0 = base + g * CHUNK_ROWS
        pltpu.sync_copy(tgt_hbm.at[pl.ds(r0, CHUNK_ROWS)], idx_t)
        for j in range(CHUNK_ROWS):
            pltpu.sync_copy(ones_v, deg_sh.at[idx_t.at[j]], add=True)
        return carry

    lax.fori_loop(0, RPW // CHUNK_ROWS, step, 0)
    plsc.subcore_barrier()

    sl = pl.ds(sid * NPT, NPT)
    pltpu.sync_copy(deg_sh.at[sl],
                    deg_out.at[pl.ds(cid * NPD + sid * NPT, NPT)])


_sc_deg = pl.kernel(
    _sc_deg_body,
    out_type=jax.ShapeDtypeStruct((NC * NPD, D), jnp.float32),
    mesh=_MESH,
    scratch_types=[
        pltpu.VMEM((CHUNK_ROWS, 128), jnp.int32),
        pltpu.VMEM((128, D), jnp.float32),
        pltpu.VMEM_SHARED((NPD, D), jnp.float32),
        pltpu.SemaphoreType.DMA,
    ],
)


def _tc_linear_body(feats_ref, agg_ref, deg_ref, wt_ref, b_ref, out_ref):
    f = feats_ref[...]
    a = agg_ref[0] + agg_ref[1]
    d = deg_ref[...]
    dg = d[0, :, 0] + d[1, :, 0]
    m = a / jnp.maximum(dg, 1.0)[:, None]
    m = jnp.where((dg > 0.0)[:, None], m, f)
    h = (jnp.dot(f, wt_ref[0:D, :], preferred_element_type=jnp.float32)
         + jnp.dot(m, wt_ref[D:2 * D, :], preferred_element_type=jnp.float32)
         + b_ref[...])
    out_ref[...] = jnp.maximum(h, 0.0)


def _tc_linear(feats, agg2, deg2, wt, b):
    blk = 1000
    grid = N // blk
    return pl.pallas_call(
        _tc_linear_body,
        grid=(grid,),
        in_specs=[
            pl.BlockSpec((blk, D), lambda i: (i, 0)),
            pl.BlockSpec((NC, blk, D), lambda i: (0, i, 0)),
            pl.BlockSpec((NC, blk, D), lambda i: (0, i, 0)),
            pl.BlockSpec((2 * D, D), lambda i: (0, 0)),
            pl.BlockSpec((1, D), lambda i: (0, 0)),
        ],
        out_specs=pl.BlockSpec((blk, D), lambda i: (i, 0)),
        out_shape=jax.ShapeDtypeStruct((N, D), jnp.float32),
    )(feats, agg2, deg2, wt, b)


def kernel(nodes, features, edge_index, W1, b1, W2, b2):
    src = edge_index[0]
    dst = edge_index[1]
    pad = E2P - E2
    tgt = jnp.concatenate([src, dst, jnp.full((pad,), N, jnp.int32)])
    nbr = jnp.concatenate([dst, src, jnp.zeros((pad,), jnp.int32)])
    tgt2 = tgt.reshape(ROWS, 128)
    nbr2 = nbr.reshape(ROWS, 128)
    zer = jnp.zeros((NPT, D), jnp.float32)
    one = jnp.ones((128, D), jnp.float32)

    deg2 = _sc_deg(tgt2, zer, one).reshape(NC, NPD, D)
    agg2 = _sc_agg(tgt2, nbr2, features, zer).reshape(NC, NP, D)
    h = _tc_linear(features, agg2, deg2, W1.T, b1.reshape(1, D))
    agg2b = _sc_agg(tgt2, nbr2, h, zer).reshape(NC, NP, D)
    return _tc_linear(h, agg2b, deg2, W2.T, b2.reshape(1, D))


# speedup vs baseline: 1.0961x; 1.0961x over previous
import jax
import jax.numpy as jnp
from jax import lax
from jax.experimental import pallas as pl
from jax.experimental.pallas import tpu as pltpu
from jax.experimental.pallas import tpu_sc as plsc

N = 10000
D = 128
E = 320000
E2 = 2 * E
NC, NS = 2, 16
NW = NC * NS
CHUNK_ROWS = 2
EPW = 20480
E2P = NW * EPW
ROWS = E2P // 128
RPW = EPW // 128
NP = 10008
NPT = 632
NPD = NS * NPT

_MESH = plsc.VectorSubcoreMesh(core_axis_name="c", subcore_axis_name="s",
                               num_cores=NC, num_subcores=NS)


R0 = 264
R1 = 56
assert 16 * (R0 + R1) == ROWS and R0 % 2 == 0 and R1 % 2 == 0


def _sc_agg_body(tgt_hbm, nbr_hbm, feats_hbm, zer_hbm, agg_out,
                 idx_t, idx_n, rows_v, agg_sh, sem):
    cid = lax.axis_index("c")
    sid = lax.axis_index("s")

    aoff = jnp.minimum(sid * NPT, NP - NPT)
    pltpu.sync_copy(zer_hbm, agg_sh.at[pl.ds(aoff, NPT)])
    plsc.subcore_barrier()

    base = jnp.where(cid == 0, sid * R0, NS * R0 + sid * R1)
    nsteps = jnp.where(cid == 0, R0 // 2, R1 // 2)

    def step(g, carry):
        r0 = base + g * 2
        pltpu.sync_copy(tgt_hbm.at[pl.ds(r0, 2)], idx_t)
        pltpu.sync_copy(nbr_hbm.at[pl.ds(r0, 2)], idx_n)
        for j in range(2):
            pltpu.async_copy(feats_hbm.at[idx_n.at[j]],
                             rows_v.at[pl.ds(j * 128, 128)], sem).wait()
        for j in range(2):
            pltpu.sync_copy(rows_v.at[pl.ds(j * 128, 128)],
                            agg_sh.at[idx_t.at[j]], add=True)
        return carry

    lax.fori_loop(0, nsteps, step, 0)
    plsc.subcore_barrier()

    pltpu.sync_copy(agg_sh.at[pl.ds(aoff, NPT)],
                    agg_out.at[pl.ds(cid * NP + aoff, NPT)])


_sc_agg = pl.kernel(
    _sc_agg_body,
    out_type=jax.ShapeDtypeStruct((NC * NP, D), jnp.float32),
    mesh=_MESH,
    scratch_types=[
        pltpu.VMEM((2, 128), jnp.int32),
        pltpu.VMEM((2, 128), jnp.int32),
        pltpu.VMEM((2 * 128, D), jnp.float32),
        pltpu.VMEM_SHARED((NP, D), jnp.float32),
        pltpu.SemaphoreType.DMA,
    ],
)


def _sc_deg_body(tgt_hbm, zer_hbm, one_hbm, deg_out,
                 idx_t, ones_v, deg_sh, sem):
    cid = lax.axis_index("c")
    sid = lax.axis_index("s")
    wid = sid * NC + cid

    pltpu.sync_copy(zer_hbm, deg_sh.at[pl.ds(sid * NPT, NPT)])
    pltpu.sync_copy(one_hbm, ones_v)
    plsc.subcore_barrier()

    base = wid * RPW

    def step(g, carry):
        r0 = base + g * CHUNK_ROWS
        pltpu.sync_copy(tgt_hbm.at[pl.ds(r0, CHUNK_ROWS)], idx_t)
        for j in range(CHUNK_ROWS):
            pltpu.sync_copy(ones_v, deg_sh.at[idx_t.at[j]], add=True)
        return carry

    lax.fori_loop(0, RPW // CHUNK_ROWS, step, 0)
    plsc.subcore_barrier()

    sl = pl.ds(sid * NPT, NPT)
    pltpu.sync_copy(deg_sh.at[sl],
                    deg_out.at[pl.ds(cid * NPD + sid * NPT, NPT)])


_sc_deg = pl.kernel(
    _sc_deg_body,
    out_type=jax.ShapeDtypeStruct((NC * NPD, D), jnp.float32),
    mesh=_MESH,
    scratch_types=[
        pltpu.VMEM((CHUNK_ROWS, 128), jnp.int32),
        pltpu.VMEM((128, D), jnp.float32),
        pltpu.VMEM_SHARED((NPD, D), jnp.float32),
        pltpu.SemaphoreType.DMA,
    ],
)


def _tc_linear_body(feats_ref, agg_ref, deg_ref, wt_ref, b_ref, out_ref):
    f = feats_ref[...]
    a = agg_ref[0] + agg_ref[1]
    d = deg_ref[...]
    dg = d[0, :, 0] + d[1, :, 0]
    m = a / jnp.maximum(dg, 1.0)[:, None]
    m = jnp.where((dg > 0.0)[:, None], m, f)
    h = (jnp.dot(f, wt_ref[0:D, :], preferred_element_type=jnp.float32)
         + jnp.dot(m, wt_ref[D:2 * D, :], preferred_element_type=jnp.float32)
         + b_ref[...])
    out_ref[...] = jnp.maximum(h, 0.0)


def _tc_linear(feats, agg2, deg2, wt, b):
    blk = 1000
    grid = N // blk
    return pl.pallas_call(
        _tc_linear_body,
        grid=(grid,),
        in_specs=[
            pl.BlockSpec((blk, D), lambda i: (i, 0)),
            pl.BlockSpec((NC, blk, D), lambda i: (0, i, 0)),
            pl.BlockSpec((NC, blk, D), lambda i: (0, i, 0)),
            pl.BlockSpec((2 * D, D), lambda i: (0, 0)),
            pl.BlockSpec((1, D), lambda i: (0, 0)),
        ],
        out_specs=pl.BlockSpec((blk, D), lambda i: (i, 0)),
        out_shape=jax.ShapeDtypeStruct((N, D), jnp.float32),
    )(feats, agg2, deg2, wt, b)


def kernel(nodes, features, edge_index, W1, b1, W2, b2):
    src = edge_index[0]
    dst = edge_index[1]
    pad = E2P - E2
    tgt = jnp.concatenate([src, dst, jnp.full((pad,), N, jnp.int32)])
    nbr = jnp.concatenate([dst, src, jnp.zeros((pad,), jnp.int32)])
    tgt2 = tgt.reshape(ROWS, 128)
    nbr2 = nbr.reshape(ROWS, 128)
    zer = jnp.zeros((NPT, D), jnp.float32)
    one = jnp.ones((128, D), jnp.float32)

    deg2 = _sc_deg(tgt2, zer, one).reshape(NC, NPD, D)
    agg2 = _sc_agg(tgt2, nbr2, features, zer).reshape(NC, NP, D)
    h = _tc_linear(features, agg2, deg2, W1.T, b1.reshape(1, D))
    agg2b = _sc_agg(tgt2, nbr2, h, zer).reshape(NC, NP, D)
    return _tc_linear(h, agg2b, deg2, W2.T, b2.reshape(1, D))


# speedup vs baseline: 1.1077x; 1.0106x over previous
import jax
import jax.numpy as jnp
from jax import lax
from jax.experimental import pallas as pl
from jax.experimental.pallas import tpu as pltpu
from jax.experimental.pallas import tpu_sc as plsc

N = 10000
D = 128
E = 320000
E2 = 2 * E
NC, NS = 2, 16
NW = NC * NS
CHUNK_ROWS = 2
EPW = 20480
E2P = NW * EPW
ROWS = E2P // 128
RPW = EPW // 128
NP = 10008
NPT = 632
NPD = NS * NPT

_MESH = plsc.VectorSubcoreMesh(core_axis_name="c", subcore_axis_name="s",
                               num_cores=NC, num_subcores=NS)


R0 = 264
R1 = 56
assert 16 * (R0 + R1) == ROWS and R0 % 2 == 0 and R1 % 2 == 0


def _sc_agg_body(tgt_hbm, nbr_hbm, feats_hbm, zer_hbm, agg_out,
                 idx_t, idx_n, rows_v, agg_sh, sem):
    cid = lax.axis_index("c")
    sid = lax.axis_index("s")

    aoff = jnp.minimum(sid * NPT, NP - NPT)
    pltpu.sync_copy(zer_hbm, agg_sh.at[pl.ds(aoff, NPT)])
    plsc.subcore_barrier()

    base = jnp.where(cid == 0, sid * R0, NS * R0 + sid * R1)
    nsteps = jnp.where(cid == 0, R0 // 4, R1 // 4)

    def step(g, carry):
        r0 = base + g * 4
        pltpu.sync_copy(tgt_hbm.at[pl.ds(r0, 4)], idx_t)
        pltpu.sync_copy(nbr_hbm.at[pl.ds(r0, 4)], idx_n)
        for h in range(2):
            for j in range(2):
                pltpu.async_copy(feats_hbm.at[idx_n.at[2 * h + j]],
                                 rows_v.at[pl.ds(j * 128, 128)], sem).wait()
            for j in range(2):
                pltpu.sync_copy(rows_v.at[pl.ds(j * 128, 128)],
                                agg_sh.at[idx_t.at[2 * h + j]], add=True)
        return carry

    lax.fori_loop(0, nsteps, step, 0)
    plsc.subcore_barrier()

    pltpu.sync_copy(agg_sh.at[pl.ds(aoff, NPT)],
                    agg_out.at[pl.ds(cid * NP + aoff, NPT)])


_sc_agg = pl.kernel(
    _sc_agg_body,
    out_type=jax.ShapeDtypeStruct((NC * NP, D), jnp.float32),
    mesh=_MESH,
    scratch_types=[
        pltpu.VMEM((4, 128), jnp.int32),
        pltpu.VMEM((4, 128), jnp.int32),
        pltpu.VMEM((2 * 128, D), jnp.float32),
        pltpu.VMEM_SHARED((NP, D), jnp.float32),
        pltpu.SemaphoreType.DMA,
    ],
)


def _sc_deg_body(tgt_hbm, zer_hbm, one_hbm, deg_out,
                 idx_t, ones_v, deg_sh, sem):
    cid = lax.axis_index("c")
    sid = lax.axis_index("s")
    wid = sid * NC + cid

    pltpu.sync_copy(zer_hbm, deg_sh.at[pl.ds(sid * NPT, NPT)])
    pltpu.sync_copy(one_hbm, ones_v)
    plsc.subcore_barrier()

    base = wid * RPW

    def step(g, carry):
        r0 = base + g * CHUNK_ROWS
        pltpu.sync_copy(tgt_hbm.at[pl.ds(r0, CHUNK_ROWS)], idx_t)
        for j in range(CHUNK_ROWS):
            pltpu.sync_copy(ones_v, deg_sh.at[idx_t.at[j]], add=True)
        return carry

    lax.fori_loop(0, RPW // CHUNK_ROWS, step, 0)
    plsc.subcore_barrier()

    sl = pl.ds(sid * NPT, NPT)
    pltpu.sync_copy(deg_sh.at[sl],
                    deg_out.at[pl.ds(cid * NPD + sid * NPT, NPT)])


_sc_deg = pl.kernel(
    _sc_deg_body,
    out_type=jax.ShapeDtypeStruct((NC * NPD, D), jnp.float32),
    mesh=_MESH,
    scratch_types=[
        pltpu.VMEM((CHUNK_ROWS, 128), jnp.int32),
        pltpu.VMEM((128, D), jnp.float32),
        pltpu.VMEM_SHARED((NPD, D), jnp.float32),
        pltpu.SemaphoreType.DMA,
    ],
)


def _tc_linear_body(feats_ref, agg_ref, deg_ref, wt_ref, b_ref, out_ref):
    f = feats_ref[...]
    a = agg_ref[0] + agg_ref[1]
    d = deg_ref[...]
    dg = d[0, :, 0] + d[1, :, 0]
    m = a / jnp.maximum(dg, 1.0)[:, None]
    m = jnp.where((dg > 0.0)[:, None], m, f)
    h = (jnp.dot(f, wt_ref[0:D, :], preferred_element_type=jnp.float32)
         + jnp.dot(m, wt_ref[D:2 * D, :], preferred_element_type=jnp.float32)
         + b_ref[...])
    out_ref[...] = jnp.maximum(h, 0.0)


def _tc_linear(feats, agg2, deg2, wt, b):
    blk = 1000
    grid = N // blk
    return pl.pallas_call(
        _tc_linear_body,
        grid=(grid,),
        in_specs=[
            pl.BlockSpec((blk, D), lambda i: (i, 0)),
            pl.BlockSpec((NC, blk, D), lambda i: (0, i, 0)),
            pl.BlockSpec((NC, blk, D), lambda i: (0, i, 0)),
            pl.BlockSpec((2 * D, D), lambda i: (0, 0)),
            pl.BlockSpec((1, D), lambda i: (0, 0)),
        ],
        out_specs=pl.BlockSpec((blk, D), lambda i: (i, 0)),
        out_shape=jax.ShapeDtypeStruct((N, D), jnp.float32),
    )(feats, agg2, deg2, wt, b)


def kernel(nodes, features, edge_index, W1, b1, W2, b2):
    src = edge_index[0]
    dst = edge_index[1]
    pad = E2P - E2
    tgt = jnp.concatenate([src, dst, jnp.full((pad,), N, jnp.int32)])
    nbr = jnp.concatenate([dst, src, jnp.zeros((pad,), jnp.int32)])
    tgt2 = tgt.reshape(ROWS, 128)
    nbr2 = nbr.reshape(ROWS, 128)
    zer = jnp.zeros((NPT, D), jnp.float32)
    one = jnp.ones((128, D), jnp.float32)

    deg2 = _sc_deg(tgt2, zer, one).reshape(NC, NPD, D)
    agg2 = _sc_agg(tgt2, nbr2, features, zer).reshape(NC, NP, D)
    h = _tc_linear(features, agg2, deg2, W1.T, b1.reshape(1, D))
    agg2b = _sc_agg(tgt2, nbr2, h, zer).reshape(NC, NP, D)
    return _tc_linear(h, agg2b, deg2, W2.T, b2.reshape(1, D))


# speedup vs baseline: 1.1175x; 1.0089x over previous
import jax
import jax.numpy as jnp
from jax import lax
from jax.experimental import pallas as pl
from jax.experimental.pallas import tpu as pltpu
from jax.experimental.pallas import tpu_sc as plsc

N = 10000
D = 128
E = 320000
E2 = 2 * E
NC, NS = 2, 16
NW = NC * NS
CHUNK_ROWS = 2
EPW = 20480
E2P = NW * EPW
ROWS = E2P // 128
RPW = EPW // 128
NP = 10008
NPT = 632
NPD = NS * NPT

_MESH = plsc.VectorSubcoreMesh(core_axis_name="c", subcore_axis_name="s",
                               num_cores=NC, num_subcores=NS)


R0 = 264
R1 = 56
assert 16 * (R0 + R1) == ROWS and R0 % 2 == 0 and R1 % 2 == 0


def _sc_agg_body(tgt_hbm, nbr_hbm, feats_hbm, zer_hbm, agg_out,
                 idx_t, idx_n, rows_v, agg_sh, sem):
    cid = lax.axis_index("c")
    sid = lax.axis_index("s")

    aoff = jnp.minimum(sid * NPT, NP - NPT)
    pltpu.sync_copy(zer_hbm, agg_sh.at[pl.ds(aoff, NPT)])
    plsc.subcore_barrier()

    base = jnp.where(cid == 0, sid * R0, NS * R0 + sid * R1)
    nsteps = jnp.where(cid == 0, R0 // 4, R1 // 4)

    def step(g, carry):
        r0 = base + g * 4
        pltpu.sync_copy(tgt_hbm.at[pl.ds(r0, 4)], idx_t)
        pltpu.sync_copy(nbr_hbm.at[pl.ds(r0, 4)], idx_n)
        for h in range(2):
            for j in range(2):
                pltpu.async_copy(feats_hbm.at[idx_n.at[2 * h + j]],
                                 rows_v.at[pl.ds(j * 128, 128)], sem).wait()
            for j in range(2):
                pltpu.sync_copy(rows_v.at[pl.ds(j * 128, 128)],
                                agg_sh.at[idx_t.at[2 * h + j]], add=True)
        return carry

    lax.fori_loop(0, nsteps, step, 0)
    plsc.subcore_barrier()

    pltpu.sync_copy(agg_sh.at[pl.ds(aoff, NPT)],
                    agg_out.at[pl.ds(cid * NP + aoff, NPT)])


_sc_agg = pl.kernel(
    _sc_agg_body,
    out_type=jax.ShapeDtypeStruct((NC * NP, D), jnp.float32),
    mesh=_MESH,
    scratch_types=[
        pltpu.VMEM((4, 128), jnp.int32),
        pltpu.VMEM((4, 128), jnp.int32),
        pltpu.VMEM((2 * 128, D), jnp.float32),
        pltpu.VMEM_SHARED((NP, D), jnp.float32),
        pltpu.SemaphoreType.DMA,
    ],
)


def _sc_deg_body(tgt_hbm, zer_hbm, one_hbm, deg_out,
                 idx_t, ones_v, deg_sh, sem):
    cid = lax.axis_index("c")
    sid = lax.axis_index("s")
    wid = sid * NC + cid

    pltpu.sync_copy(zer_hbm, deg_sh.at[pl.ds(sid * NPT, NPT)])
    pltpu.sync_copy(one_hbm, ones_v)
    plsc.subcore_barrier()

    base = wid * RPW

    def step(g, carry):
        r0 = base + g * 4
        pltpu.sync_copy(tgt_hbm.at[pl.ds(r0, 4)], idx_t)
        for j in range(4):
            pltpu.sync_copy(ones_v, deg_sh.at[idx_t.at[j]], add=True)
        return carry

    lax.fori_loop(0, RPW // 4, step, 0)
    plsc.subcore_barrier()

    sl = pl.ds(sid * NPT, NPT)
    pltpu.sync_copy(deg_sh.at[sl],
                    deg_out.at[pl.ds(cid * NPD + sid * NPT, NPT)])


_sc_deg = pl.kernel(
    _sc_deg_body,
    out_type=jax.ShapeDtypeStruct((NC * NPD, D), jnp.float32),
    mesh=_MESH,
    scratch_types=[
        pltpu.VMEM((4, 128), jnp.int32),
        pltpu.VMEM((128, D), jnp.float32),
        pltpu.VMEM_SHARED((NPD, D), jnp.float32),
        pltpu.SemaphoreType.DMA,
    ],
)


def _tc_linear_body(feats_ref, agg_ref, deg_ref, wt_ref, b_ref, out_ref):
    f = feats_ref[...]
    a = agg_ref[0] + agg_ref[1]
    d = deg_ref[...]
    dg = d[0, :, 0] + d[1, :, 0]
    m = a / jnp.maximum(dg, 1.0)[:, None]
    m = jnp.where((dg > 0.0)[:, None], m, f)
    h = (jnp.dot(f, wt_ref[0:D, :], preferred_element_type=jnp.float32)
         + jnp.dot(m, wt_ref[D:2 * D, :], preferred_element_type=jnp.float32)
         + b_ref[...])
    out_ref[...] = jnp.maximum(h, 0.0)


def _tc_linear(feats, agg2, deg2, wt, b):
    blk = 1000
    grid = N // blk
    return pl.pallas_call(
        _tc_linear_body,
        grid=(grid,),
        in_specs=[
            pl.BlockSpec((blk, D), lambda i: (i, 0)),
            pl.BlockSpec((NC, blk, D), lambda i: (0, i, 0)),
            pl.BlockSpec((NC, blk, D), lambda i: (0, i, 0)),
            pl.BlockSpec((2 * D, D), lambda i: (0, 0)),
            pl.BlockSpec((1, D), lambda i: (0, 0)),
        ],
        out_specs=pl.BlockSpec((blk, D), lambda i: (i, 0)),
        out_shape=jax.ShapeDtypeStruct((N, D), jnp.float32),
    )(feats, agg2, deg2, wt, b)


def kernel(nodes, features, edge_index, W1, b1, W2, b2):
    src = edge_index[0]
    dst = edge_index[1]
    pad = E2P - E2
    tgt = jnp.concatenate([src, dst, jnp.full((pad,), N, jnp.int32)])
    nbr = jnp.concatenate([dst, src, jnp.zeros((pad,), jnp.int32)])
    tgt2 = tgt.reshape(ROWS, 128)
    nbr2 = nbr.reshape(ROWS, 128)
    zer = jnp.zeros((NPT, D), jnp.float32)
    one = jnp.ones((128, D), jnp.float32)

    deg2 = _sc_deg(tgt2, zer, one).reshape(NC, NPD, D)
    agg2 = _sc_agg(tgt2, nbr2, features, zer).reshape(NC, NP, D)
    h = _tc_linear(features, agg2, deg2, W1.T, b1.reshape(1, D))
    agg2b = _sc_agg(tgt2, nbr2, h, zer).reshape(NC, NP, D)
    return _tc_linear(h, agg2b, deg2, W2.T, b2.reshape(1, D))
